# Initial kernel scaffold; baseline (speedup 1.0000x reference)
import jax
import jax.numpy as jnp
from jax.experimental import pallas as pl


def kernel(x, edge_index, edge_attr, emb, W1, We1, as1, ad1, ae1, b1, W2, We2, as2, ad2, ae2, b2, M1, c1, M2, c2):
    raise NotImplementedError("write your pallas kernel here")


# speedup vs baseline: 12.1049x; 12.1049x over previous
import functools

import jax
import jax.numpy as jnp
from jax import lax
from jax.experimental import pallas as pl
from jax.experimental.pallas import tpu as pltpu
from jax.experimental.pallas import tpu_sc as plsc

N = 10000
E = 160000
D = 128
HID = 64
HEADS = 8
DE = 16

NC = 2
NS = 16
NW = NC * NS
CHUNK = E // NW
IDXW = 125
IDXR = 8
SUB = IDXW * IDXR
NSUB = CHUNK // SUB
NROWA = 624
TAIL = N - NS * NROWA
TAIL0 = NS * NROWA

_f32 = jnp.float32
_i32 = jnp.int32


def _node1_body(x_ref, emb_ref, w1_ref, w1sd_ref, xh1_ref, sd1_ref):
    h = pl.program_id(1)
    ids = x_ref[...]
    iota = lax.broadcasted_iota(_i32, (ids.shape[0], 32), 1)
    onehot = (ids == iota).astype(_f32)
    h0 = jnp.dot(onehot, emb_ref[...], preferred_element_type=_f32)
    xh1_ref[0] = jnp.dot(h0, w1_ref[0], preferred_element_type=_f32)

    @pl.when(h == 0)
    def _():
        sd1_ref[...] = jnp.dot(h0, w1sd_ref[...], preferred_element_type=_f32)


def _tc_node1(x, emb, w1hm, w1sd):
    bn = 1000
    return pl.pallas_call(
        _node1_body,
        grid=(N // bn, HEADS),
        in_specs=[
            pl.BlockSpec((bn, 1), lambda nt, h: (nt, 0)),
            pl.BlockSpec((32, D), lambda nt, h: (0, 0)),
            pl.BlockSpec((1, D, HID), lambda nt, h: (h, 0, 0)),
            pl.BlockSpec((D, 16), lambda nt, h: (0, 0)),
        ],
        out_specs=[
            pl.BlockSpec((1, bn, HID), lambda nt, h: (h, nt, 0)),
            pl.BlockSpec((bn, 16), lambda nt, h: (nt, 0)),
        ],
        out_shape=[
            jax.ShapeDtypeStruct((HEADS, N, HID), _f32),
            jax.ShapeDtypeStruct((N, 16), _f32),
        ],
    )(x, emb, w1hm, w1sd)


def _eatt_body(ea_ref, wae_ref, out_ref):
    out_ref[...] = jnp.dot(ea_ref[...], wae_ref[...],
                           preferred_element_type=_f32)


def _tc_eatt(ea, wae):
    be = 2000
    return pl.pallas_call(
        _eatt_body,
        grid=(E // be,),
        in_specs=[
            pl.BlockSpec((be, DE), lambda i: (i, 0)),
            pl.BlockSpec((DE, 16), lambda i: (0, 0)),
        ],
        out_specs=pl.BlockSpec((be, 16), lambda i: (i, 0)),
        out_shape=jax.ShapeDtypeStruct((E, 16), _f32),
    )(ea, wae)


def _h1xh2_body(p0_ref, p1_ref, d0_ref, d1_ref, b1_ref, w2_ref, w2sd_ref,
                xh2_ref, sd2_ref):
    h = pl.program_id(1)
    den = d0_ref[0] + d1_ref[0] + 1e-16
    lane = lax.broadcasted_iota(_i32, den.shape, 1)
    dh = jnp.sum(jnp.where(lane == h, den, 0.0), axis=1, keepdims=True)
    hb = (p0_ref[0, 0] + p1_ref[0, 0]) / dh + b1_ref[0]
    hb = jnp.where(hb > 0, hb, jnp.exp(hb) - 1.0)
    dx = jnp.dot(hb, w2_ref[0], preferred_element_type=_f32)
    ds = jnp.dot(hb, w2sd_ref[0], preferred_element_type=_f32)

    @pl.when(h == 0)
    def _():
        xh2_ref[...] = jnp.zeros_like(xh2_ref)
        sd2_ref[...] = jnp.zeros_like(sd2_ref)

    xh2_ref[...] += dx
    sd2_ref[...] += ds


def _tc_h1xh2(out1p, den1, b1hm, w2hm, w2sdhm):
    bn = 1000
    return pl.pallas_call(
        _h1xh2_body,
        grid=(N // bn, HEADS),
        in_specs=[
            pl.BlockSpec((1, 1, bn, HID), lambda nt, h: (0, h, nt, 0)),
            pl.BlockSpec((1, 1, bn, HID), lambda nt, h: (1, h, nt, 0)),
            pl.BlockSpec((1, bn, 16), lambda nt, h: (0, nt, 0)),
            pl.BlockSpec((1, bn, 16), lambda nt, h: (1, nt, 0)),
            pl.BlockSpec((1, 1, HID), lambda nt, h: (h, 0, 0)),
            pl.BlockSpec((1, HID, HID), lambda nt, h: (h, 0, 0)),
            pl.BlockSpec((1, HID, 16), lambda nt, h: (h, 0, 0)),
        ],
        out_specs=[
            pl.BlockSpec((bn, HID), lambda nt, h: (nt, 0)),
            pl.BlockSpec((bn, 16), lambda nt, h: (nt, 0)),
        ],
        out_shape=[
            jax.ShapeDtypeStruct((N, HID), _f32),
            jax.ShapeDtypeStruct((N, 16), _f32),
        ],
    )(out1p, out1p, den1, den1, b1hm, w2hm, w2sdhm)


def _ab_body(p0_ref, p1_ref, d0_ref, d1_ref, b2_ref, m1a_ref, m1b_ref,
             ab_ref):
    den = d0_ref[0] + d1_ref[0] + 1e-16
    h2 = (p0_ref[0, 0] + p1_ref[0, 0]) / den[:, 0:1] + b2_ref[...]
    h2 = jnp.where(h2 > 0, h2, jnp.exp(h2) - 1.0)
    ab_ref[0] = jnp.dot(h2, m1a_ref[...], preferred_element_type=_f32)
    ab_ref[1] = jnp.dot(h2, m1b_ref[...], preferred_element_type=_f32)


def _tc_ab(out2p, den2, b2, m1a, m1b):
    bn = 1000
    return pl.pallas_call(
        _ab_body,
        grid=(N // bn,),
        in_specs=[
            pl.BlockSpec((1, 1, bn, HID), lambda nt: (0, 0, nt, 0)),
            pl.BlockSpec((1, 1, bn, HID), lambda nt: (1, 0, nt, 0)),
            pl.BlockSpec((1, bn, 16), lambda nt: (0, nt, 0)),
            pl.BlockSpec((1, bn, 16), lambda nt: (1, nt, 0)),
            pl.BlockSpec((1, HID), lambda nt: (0, 0)),
            pl.BlockSpec((HID, HID), lambda nt: (0, 0)),
            pl.BlockSpec((HID, HID), lambda nt: (0, 0)),
        ],
        out_specs=pl.BlockSpec((2, bn, HID), lambda nt: (0, nt, 0)),
        out_shape=jax.ShapeDtypeStruct((2, N, HID), _f32),
    )(out2p, out2p, den2, den2, b2.reshape(1, HID), m1a, m1b)


def _mlp_body(r_ref, m2_ref, c2_ref, out_ref):
    out_ref[...] = jnp.dot(r_ref[...], m2_ref[...],
                           preferred_element_type=_f32) + c2_ref[...]


def _tc_mlp(r, m2p, c2p):
    be = 2000
    return pl.pallas_call(
        _mlp_body,
        grid=(E // be,),
        in_specs=[
            pl.BlockSpec((be, HID), lambda i: (i, 0)),
            pl.BlockSpec((HID, 8), lambda i: (0, 0)),
            pl.BlockSpec((1, 8), lambda i: (0, 0)),
        ],
        out_specs=pl.BlockSpec((be, 8), lambda i: (i, 0)),
        out_shape=jax.ShapeDtypeStruct((E, 8), _f32),
    )(r, m2p, c2p.reshape(1, 8))


_MESH = plsc.VectorSubcoreMesh(core_axis_name="c", subcore_axis_name="s")
_SCPARAMS = pltpu.CompilerParams(use_tc_tiling_on_sc=False)


def _worker():
    cid = lax.axis_index("c")
    tid = lax.axis_index("s")
    return cid, tid, cid * NS + tid


def _m8(v):
    return pl.multiple_of(v, 8)


def _att_body(src_ref, dst_ref, sd_ref, eatt_ref, ex_ref,
              srcv, dstv, gsv, gdv, gev, exv, sem, *, layer):
    cid, tid, wid = _worker()
    lane = lax.iota(_i32, 16)
    if layer == 1:
        perm_s = None
        perm_d = jnp.minimum(lane + 8, 15)
        perm_e = None
        nlanes = 8
    else:
        perm_s = lane * 0
        perm_d = lane * 0 + 1
        perm_e = lane * 0 + 8
        nlanes = 1

    def _take(vec, perm):
        if perm is None:
            return vec
        return vec.at[perm].get(mode="promise_in_bounds")

    for sub in range(NSUB):
        base = _m8(wid * CHUNK + sub * SUB)
        rbase = _m8(base // IDXW)
        pltpu.sync_copy(src_ref.at[pl.ds(rbase, IDXR)], srcv)
        pltpu.sync_copy(dst_ref.at[pl.ds(rbase, IDXR)], dstv)
        for j in range(IDXR):
            pltpu.async_copy(sd_ref.at[srcv.at[j]],
                             gsv.at[pl.ds(j * IDXW, IDXW)], sem).wait()
            pltpu.async_copy(sd_ref.at[dstv.at[j]],
                             gdv.at[pl.ds(j * IDXW, IDXW)], sem).wait()
        pltpu.sync_copy(eatt_ref.at[pl.ds(base, SUB)], gev)

        def row(i, _):
            vs = _take(gsv[i, :], perm_s)
            vd = _take(gdv[i, :], perm_d)
            ve = _take(gev[i, :], perm_e)
            a = vs + vd + ve
            a = jnp.where(a > 0, a, 0.2 * a)
            e = jnp.exp(a)
            exv[i, :] = jnp.where(lane < nlanes, e, 0.0)
            return _

        lax.fori_loop(0, SUB, row, None)
        pltpu.sync_copy(exv, ex_ref.at[pl.ds(base, SUB)])


def _sc_att(src2d, dst2d, sd, eatt, layer):
    f = pl.kernel(
        functools.partial(_att_body, layer=layer),
        out_type=jax.ShapeDtypeStruct((E, 16), _f32),
        mesh=_MESH,
        compiler_params=_SCPARAMS,
        scratch_types=[
            pltpu.VMEM((IDXR, IDXW), _i32),
            pltpu.VMEM((IDXR, IDXW), _i32),
            pltpu.VMEM((SUB, 16), _f32),
            pltpu.VMEM((SUB, 16), _f32),
            pltpu.VMEM((SUB, 16), _f32),
            pltpu.VMEM((SUB, 16), _f32),
            pltpu.SemaphoreType.DMA,
        ],
    )
    return f(src2d, dst2d, sd, eatt)


def _zero_slice(zeros_ref, acc, tid):
    pltpu.sync_copy(zeros_ref.at[pl.ds(_m8(tid * NROWA), NROWA)],
                    acc.at[pl.ds(_m8(tid * NROWA), NROWA)])

    @pl.when(tid == 0)
    def _():
        pltpu.sync_copy(zeros_ref.at[pl.ds(TAIL0, TAIL)],
                        acc.at[pl.ds(TAIL0, TAIL)])


def _agg_body(gidx_ref, dst_ref, ex_ref, tab_ref, z16_ref, z64_ref,
              den_ref, out_ref,
              gv, dstv, gxh, exv, accd, accm, sem, *, nheads):
    cid, tid, wid = _worker()

    _zero_slice(z16_ref, accd, tid)
    plsc.subcore_barrier()
    for sub in range(NSUB):
        base = _m8(wid * CHUNK + sub * SUB)
        rbase = _m8(base // IDXW)
        pltpu.sync_copy(dst_ref.at[pl.ds(rbase, IDXR)], dstv)
        pltpu.sync_copy(ex_ref.at[pl.ds(base, SUB)], exv)
        for j in range(IDXR):
            pltpu.sync_copy(exv.at[pl.ds(j * IDXW, IDXW)],
                            accd.at[dstv.at[j]], add=True)
    plsc.subcore_barrier()
    pltpu.sync_copy(accd.at[pl.ds(_m8(tid * NROWA), NROWA)],
                    den_ref.at[cid, pl.ds(_m8(tid * NROWA), NROWA)])

    @pl.when(tid == 0)
    def _():
        pltpu.sync_copy(accd.at[pl.ds(TAIL0, TAIL)],
                        den_ref.at[cid, pl.ds(TAIL0, TAIL)])

    def head_body(h, _):
        _zero_slice(z64_ref, accm, tid)
        plsc.subcore_barrier()
        hsplat = jnp.full((16,), h, _i32)

        def sub_body(sub, _):
            base = _m8(wid * CHUNK + sub * SUB)
            rbase = _m8(wid * (CHUNK // IDXW) + sub * (SUB // IDXW))
            pltpu.sync_copy(gidx_ref.at[h, pl.ds(rbase, IDXR)], gv)
            pltpu.sync_copy(dst_ref.at[pl.ds(rbase, IDXR)], dstv)
            pltpu.sync_copy(ex_ref.at[pl.ds(base, SUB)], exv)

            def jrow(j, _):
                pltpu.async_copy(tab_ref.at[gv.at[j]], gxh, sem).wait()

                def rowm(i, _):
                    vex = exv[j * IDXW + i, :].at[hsplat].get(
                        mode="promise_in_bounds")
                    for k in range(HID // 16):
                        s = pl.ds(k * 16, 16)
                        gxh[i, s] = gxh[i, s] * vex
                    return _

                lax.fori_loop(0, IDXW, rowm, None)
                pltpu.sync_copy(gxh, accm.at[dstv.at[j]], add=True)
                return _

            lax.fori_loop(0, IDXR, jrow, None)
            return _

        lax.fori_loop(0, NSUB, sub_body, None)
        plsc.subcore_barrier()
        pltpu.sync_copy(accm.at[pl.ds(_m8(tid * NROWA), NROWA)],
                        out_ref.at[cid, pl.ds(_m8(h * N + tid * NROWA), NROWA)])

        @pl.when(tid == 0)
        def _():
            pltpu.sync_copy(accm.at[pl.ds(TAIL0, TAIL)],
                            out_ref.at[cid, pl.ds(h * N + TAIL0, TAIL)])

        plsc.subcore_barrier()
        return _

    lax.fori_loop(0, nheads, head_body, None)


def _sc_agg(gidx3d, dst2d, ex, table, zeros16, zeros64, nheads):
    f = pl.kernel(
        functools.partial(_agg_body, nheads=nheads),
        out_type=[
            jax.ShapeDtypeStruct((NC, N, 16), _f32),
            jax.ShapeDtypeStruct((NC, nheads * N, HID), _f32),
        ],
        mesh=_MESH,
        compiler_params=_SCPARAMS,
        scratch_types=[
            pltpu.VMEM((IDXR, IDXW), _i32),
            pltpu.VMEM((IDXR, IDXW), _i32),
            pltpu.VMEM((IDXW, HID), _f32),
            pltpu.VMEM((SUB, 16), _f32),
            pltpu.VMEM_SHARED((N, 16), _f32),
            pltpu.VMEM_SHARED((N, HID), _f32),
            pltpu.SemaphoreType.DMA,
        ],
    )
    return f(gidx3d, dst2d, ex, table, zeros16, zeros64)


def _mlpg_body(src_ref, dstn_ref, ab_ref, c1_ref, r_ref,
               sv, dv, ga, gb, c1v, sem):
    cid, tid, wid = _worker()
    pltpu.sync_copy(c1_ref, c1v)

    for sub in range(NSUB):
        base = _m8(wid * CHUNK + sub * SUB)
        rbase = _m8(base // IDXW)
        pltpu.sync_copy(src_ref.at[pl.ds(rbase, IDXR)], sv)
        pltpu.sync_copy(dstn_ref.at[pl.ds(rbase, IDXR)], dv)
        for j in range(IDXR):
            pltpu.async_copy(ab_ref.at[sv.at[j]],
                             ga.at[pl.ds(j * IDXW, IDXW)], sem).wait()
        for half in range(2):
            for j2 in range(IDXR // 2):
                j = half * (IDXR // 2) + j2
                pltpu.async_copy(ab_ref.at[dv.at[j]],
                                 gb.at[pl.ds(j2 * IDXW, IDXW)], sem).wait()
            hbase = half * (SUB // 2)

            def row(i, _):
                for k in range(HID // 16):
                    s = pl.ds(k * 16, 16)
                    v = ga[hbase + i, s] + gb[i, s] + c1v[s]
                    ga[hbase + i, s] = jnp.maximum(v, 0.0)
                return _

            lax.fori_loop(0, SUB // 2, row, None)
        pltpu.sync_copy(ga, r_ref.at[pl.ds(base, SUB)])


def _sc_mlpg(src2d, dstn2d, ab_flat, c1):
    f = pl.kernel(
        _mlpg_body,
        out_type=jax.ShapeDtypeStruct((E, HID), _f32),
        mesh=_MESH,
        compiler_params=_SCPARAMS,
        scratch_types=[
            pltpu.VMEM((IDXR, IDXW), _i32),
            pltpu.VMEM((IDXR, IDXW), _i32),
            pltpu.VMEM((SUB, HID), _f32),
            pltpu.VMEM((SUB // 2, HID), _f32),
            pltpu.VMEM((HID,), _f32),
            pltpu.SemaphoreType.DMA,
        ],
    )
    return f(src2d, dstn2d, ab_flat, c1)


def kernel(x, edge_index, edge_attr, emb, W1, We1, as1, ad1, ae1, b1,
           W2, We2, as2, ad2, ae2, b2, M1, c1, M2, c2):
    src = edge_index[0]
    dst = edge_index[1]
    src2d = src.reshape(E // IDXW, IDXW)
    dst2d = dst.reshape(E // IDXW, IDXW)

    w1r = W1.reshape(D, HEADS, HID)
    W1sd = jnp.concatenate([
        jnp.einsum("dhc,hc->dh", w1r, as1),
        jnp.einsum("dhc,hc->dh", w1r, ad1)], axis=1)
    Wae = jnp.zeros((DE, 16), _f32)
    Wae = Wae.at[:, 0:HEADS].set(
        jnp.einsum("dhc,hc->dh", We1.reshape(DE, HEADS, HID), ae1))
    Wae = Wae.at[:, 8].set(We2 @ ae2[0])
    W2sd = jnp.zeros((HEADS * HID, 16), _f32)
    W2sd = W2sd.at[:, 0].set(W2 @ as2[0]).at[:, 1].set(W2 @ ad2[0])
    w1hm = w1r.transpose(1, 0, 2)
    w2hm = W2.reshape(HEADS, HID, HID)
    w2sdhm = W2sd.reshape(HEADS, HID, 16)
    b1hm = b1.reshape(HEADS, 1, HID)
    m2p = jnp.zeros((HID, 8), _f32).at[:, 0:3].set(M2)
    c2p = jnp.zeros((8,), _f32).at[0:3].set(c2)

    heads_off = (jnp.arange(HEADS, dtype=_i32) * N)[:, None, None]
    gidx1 = (src2d[None] + heads_off).astype(_i32)
    gidx2 = src2d[None]
    dstn2d = dst2d + N

    zeros16 = jnp.zeros((N, 16), _f32)
    zeros64 = jnp.zeros((N, HID), _f32)

    xh1hm, sd1 = _tc_node1(x.astype(_i32), emb, w1hm, W1sd)
    eatt = _tc_eatt(edge_attr, Wae)
    ex1 = _sc_att(src2d, dst2d, sd1, eatt, 1)
    den1, out1p = _sc_agg(gidx1, dst2d, ex1,
                          xh1hm.reshape(HEADS * N, HID), zeros16, zeros64,
                          HEADS)
    out1p = out1p.reshape(NC, HEADS, N, HID)

    xh2, sd2 = _tc_h1xh2(out1p, den1, b1hm, w2hm, w2sdhm)
    ex2 = _sc_att(src2d, dst2d, sd2, eatt, 2)
    den2, out2p = _sc_agg(gidx2, dst2d, ex2, xh2, zeros16, zeros64, 1)
    out2p = out2p.reshape(NC, 1, N, HID)

    ab = _tc_ab(out2p, den2, b2, M1[:HID], M1[HID:])
    r = _sc_mlpg(src2d, dstn2d, ab.reshape(2 * N, HID), c1)
    out = _tc_mlp(r, m2p, c2p)
    return out[:, 0:3]


# speedup vs baseline: 13.2318x; 1.0931x over previous
import functools

import jax
import jax.numpy as jnp
from jax import lax
from jax.experimental import pallas as pl
from jax.experimental.pallas import tpu as pltpu
from jax.experimental.pallas import tpu_sc as plsc

N = 10000
E = 160000
D = 128
HID = 64
HEADS = 8
DE = 16

NC = 2
NS = 16
NW = NC * NS
CHUNK = E // NW
IDXW = 125
IDXR = 8
SUB = IDXW * IDXR
NSUB = CHUNK // SUB
NROWA = 624
TAIL = N - NS * NROWA
TAIL0 = NS * NROWA

_f32 = jnp.float32
_i32 = jnp.int32


def _node1_body(x_ref, emb_ref, w1_ref, w1sd_ref, xh1_ref, sd1_ref):
    h = pl.program_id(1)
    ids = x_ref[...]
    iota = lax.broadcasted_iota(_i32, (ids.shape[0], 32), 1)
    onehot = (ids == iota).astype(_f32)
    h0 = jnp.dot(onehot, emb_ref[...], preferred_element_type=_f32)
    xh1_ref[0] = jnp.dot(h0, w1_ref[0], preferred_element_type=_f32)

    @pl.when(h == 0)
    def _():
        sd1_ref[...] = jnp.dot(h0, w1sd_ref[...], preferred_element_type=_f32)


def _tc_node1(x, emb, w1hm, w1sd):
    bn = 1000
    return pl.pallas_call(
        _node1_body,
        grid=(N // bn, HEADS),
        in_specs=[
            pl.BlockSpec((bn, 1), lambda nt, h: (nt, 0)),
            pl.BlockSpec((32, D), lambda nt, h: (0, 0)),
            pl.BlockSpec((1, D, HID), lambda nt, h: (h, 0, 0)),
            pl.BlockSpec((D, 16), lambda nt, h: (0, 0)),
        ],
        out_specs=[
            pl.BlockSpec((1, bn, HID), lambda nt, h: (h, nt, 0)),
            pl.BlockSpec((bn, 16), lambda nt, h: (nt, 0)),
        ],
        out_shape=[
            jax.ShapeDtypeStruct((HEADS, N, HID), _f32),
            jax.ShapeDtypeStruct((N, 16), _f32),
        ],
    )(x, emb, w1hm, w1sd)


def _eatt_body(ea_ref, wae_ref, out_ref):
    out_ref[...] = jnp.dot(ea_ref[...], wae_ref[...],
                           preferred_element_type=_f32)


def _tc_eatt(ea, wae):
    be = 2000
    return pl.pallas_call(
        _eatt_body,
        grid=(E // be,),
        in_specs=[
            pl.BlockSpec((be, DE), lambda i: (i, 0)),
            pl.BlockSpec((DE, 16), lambda i: (0, 0)),
        ],
        out_specs=pl.BlockSpec((be, 16), lambda i: (i, 0)),
        out_shape=jax.ShapeDtypeStruct((E, 16), _f32),
    )(ea, wae)


def _h1xh2_body(p0_ref, p1_ref, d0_ref, d1_ref, b1_ref, w2_ref, w2sd_ref,
                xh2_ref, sd2_ref):
    h = pl.program_id(1)
    den = d0_ref[0] + d1_ref[0] + 1e-16
    lane = lax.broadcasted_iota(_i32, den.shape, 1)
    dh = jnp.sum(jnp.where(lane == h, den, 0.0), axis=1, keepdims=True)
    hb = (p0_ref[0, 0] + p1_ref[0, 0]) / dh + b1_ref[0]
    hb = jnp.where(hb > 0, hb, jnp.exp(hb) - 1.0)
    dx = jnp.dot(hb, w2_ref[0], preferred_element_type=_f32)
    ds = jnp.dot(hb, w2sd_ref[0], preferred_element_type=_f32)

    @pl.when(h == 0)
    def _():
        xh2_ref[...] = jnp.zeros_like(xh2_ref)
        sd2_ref[...] = jnp.zeros_like(sd2_ref)

    xh2_ref[...] += dx
    sd2_ref[...] += ds


def _tc_h1xh2(out1p, den1, b1hm, w2hm, w2sdhm):
    bn = 1000
    return pl.pallas_call(
        _h1xh2_body,
        grid=(N // bn, HEADS),
        in_specs=[
            pl.BlockSpec((1, 1, bn, HID), lambda nt, h: (0, h, nt, 0)),
            pl.BlockSpec((1, 1, bn, HID), lambda nt, h: (1, h, nt, 0)),
            pl.BlockSpec((1, bn, 16), lambda nt, h: (0, nt, 0)),
            pl.BlockSpec((1, bn, 16), lambda nt, h: (1, nt, 0)),
            pl.BlockSpec((1, 1, HID), lambda nt, h: (h, 0, 0)),
            pl.BlockSpec((1, HID, HID), lambda nt, h: (h, 0, 0)),
            pl.BlockSpec((1, HID, 16), lambda nt, h: (h, 0, 0)),
        ],
        out_specs=[
            pl.BlockSpec((bn, HID), lambda nt, h: (nt, 0)),
            pl.BlockSpec((bn, 16), lambda nt, h: (nt, 0)),
        ],
        out_shape=[
            jax.ShapeDtypeStruct((N, HID), _f32),
            jax.ShapeDtypeStruct((N, 16), _f32),
        ],
    )(out1p, out1p, den1, den1, b1hm, w2hm, w2sdhm)


def _ab_body(p0_ref, p1_ref, d0_ref, d1_ref, b2_ref, m1a_ref, m1b_ref,
             ab_ref):
    den = d0_ref[0] + d1_ref[0] + 1e-16
    h2 = (p0_ref[0, 0] + p1_ref[0, 0]) / den[:, 0:1] + b2_ref[...]
    h2 = jnp.where(h2 > 0, h2, jnp.exp(h2) - 1.0)
    ab_ref[0] = jnp.dot(h2, m1a_ref[...], preferred_element_type=_f32)
    ab_ref[1] = jnp.dot(h2, m1b_ref[...], preferred_element_type=_f32)


def _tc_ab(out2p, den2, b2, m1a, m1b):
    bn = 1000
    return pl.pallas_call(
        _ab_body,
        grid=(N // bn,),
        in_specs=[
            pl.BlockSpec((1, 1, bn, HID), lambda nt: (0, 0, nt, 0)),
            pl.BlockSpec((1, 1, bn, HID), lambda nt: (1, 0, nt, 0)),
            pl.BlockSpec((1, bn, 16), lambda nt: (0, nt, 0)),
            pl.BlockSpec((1, bn, 16), lambda nt: (1, nt, 0)),
            pl.BlockSpec((1, HID), lambda nt: (0, 0)),
            pl.BlockSpec((HID, HID), lambda nt: (0, 0)),
            pl.BlockSpec((HID, HID), lambda nt: (0, 0)),
        ],
        out_specs=pl.BlockSpec((2, bn, HID), lambda nt: (0, nt, 0)),
        out_shape=jax.ShapeDtypeStruct((2, N, HID), _f32),
    )(out2p, out2p, den2, den2, b2.reshape(1, HID), m1a, m1b)


def _mlp_body(r_ref, m2_ref, c2_ref, out_ref):
    out_ref[...] = jnp.dot(r_ref[...], m2_ref[...],
                           preferred_element_type=_f32) + c2_ref[...]


def _tc_mlp(r, m2p, c2p):
    be = 2000
    return pl.pallas_call(
        _mlp_body,
        grid=(E // be,),
        in_specs=[
            pl.BlockSpec((be, HID), lambda i: (i, 0)),
            pl.BlockSpec((HID, 8), lambda i: (0, 0)),
            pl.BlockSpec((1, 8), lambda i: (0, 0)),
        ],
        out_specs=pl.BlockSpec((be, 8), lambda i: (i, 0)),
        out_shape=jax.ShapeDtypeStruct((E, 8), _f32),
    )(r, m2p, c2p.reshape(1, 8))


_MESH = plsc.VectorSubcoreMesh(core_axis_name="c", subcore_axis_name="s")
_SCPARAMS = pltpu.CompilerParams(use_tc_tiling_on_sc=False)


def _worker():
    cid = lax.axis_index("c")
    tid = lax.axis_index("s")
    return cid, tid, cid * NS + tid


def _m8(v):
    return pl.multiple_of(v, 8)


def _att_body(src_ref, dst_ref, sd_ref, eatt_ref, ex_ref,
              srcv, dstv, gsv, gdv, gev, exv, sem, sem2, *, layer):
    cid, tid, wid = _worker()
    lane = lax.iota(_i32, 16)
    if layer == 1:
        perm_s = None
        perm_d = jnp.minimum(lane + 8, 15)
        perm_e = None
        nlanes = 8
    else:
        perm_s = lane * 0
        perm_d = lane * 0 + 1
        perm_e = lane * 0 + 8
        nlanes = 1

    def _take(vec, perm):
        if perm is None:
            return vec
        return vec.at[perm].get(mode="promise_in_bounds")

    for sub in range(NSUB):
        base = _m8(wid * CHUNK + sub * SUB)
        rbase = _m8(base // IDXW)
        pltpu.sync_copy(src_ref.at[pl.ds(rbase, IDXR)], srcv)
        pltpu.sync_copy(dst_ref.at[pl.ds(rbase, IDXR)], dstv)
        for j in range(IDXR):
            ca = pltpu.async_copy(sd_ref.at[srcv.at[j]],
                                  gsv.at[pl.ds(j * IDXW, IDXW)], sem)
            cb = pltpu.async_copy(sd_ref.at[dstv.at[j]],
                                  gdv.at[pl.ds(j * IDXW, IDXW)], sem2)
            ca.wait()
            cb.wait()
        pltpu.sync_copy(eatt_ref.at[pl.ds(base, SUB)], gev)

        def row(i, _):
            vs = _take(gsv[i, :], perm_s)
            vd = _take(gdv[i, :], perm_d)
            ve = _take(gev[i, :], perm_e)
            a = vs + vd + ve
            a = jnp.where(a > 0, a, 0.2 * a)
            e = jnp.exp(a)
            exv[i, :] = jnp.where(lane < nlanes, e, 0.0)
            return _

        lax.fori_loop(0, SUB, row, None)
        pltpu.sync_copy(exv, ex_ref.at[pl.ds(base, SUB)])


def _sc_att(src2d, dst2d, sd, eatt, layer):
    f = pl.kernel(
        functools.partial(_att_body, layer=layer),
        out_type=jax.ShapeDtypeStruct((E, 16), _f32),
        mesh=_MESH,
        compiler_params=_SCPARAMS,
        scratch_types=[
            pltpu.VMEM((IDXR, IDXW), _i32),
            pltpu.VMEM((IDXR, IDXW), _i32),
            pltpu.VMEM((SUB, 16), _f32),
            pltpu.VMEM((SUB, 16), _f32),
            pltpu.VMEM((SUB, 16), _f32),
            pltpu.VMEM((SUB, 16), _f32),
            pltpu.SemaphoreType.DMA,
            pltpu.SemaphoreType.DMA,
        ],
    )
    return f(src2d, dst2d, sd, eatt)


def _zero_slice(zeros_ref, acc, tid):
    pltpu.sync_copy(zeros_ref.at[pl.ds(_m8(tid * NROWA), NROWA)],
                    acc.at[pl.ds(_m8(tid * NROWA), NROWA)])

    @pl.when(tid == 0)
    def _():
        pltpu.sync_copy(zeros_ref.at[pl.ds(TAIL0, TAIL)],
                        acc.at[pl.ds(TAIL0, TAIL)])


def _agg_body(gidx_ref, dst_ref, ex_ref, tab_ref, z16_ref, z64_ref,
              den_ref, out_ref,
              gv, dstv, gxh, exv, accd, accm, sem, sem2, *, nheads):
    cid, tid, wid = _worker()

    _zero_slice(z16_ref, accd, tid)
    plsc.subcore_barrier()
    for sub in range(NSUB):
        base = _m8(wid * CHUNK + sub * SUB)
        rbase = _m8(base // IDXW)
        pltpu.sync_copy(dst_ref.at[pl.ds(rbase, IDXR)], dstv)
        pltpu.sync_copy(ex_ref.at[pl.ds(base, SUB)], exv)
        for j in range(IDXR):
            pltpu.sync_copy(exv.at[pl.ds(j * IDXW, IDXW)],
                            accd.at[dstv.at[j]], add=True)
    plsc.subcore_barrier()
    pltpu.sync_copy(accd.at[pl.ds(_m8(tid * NROWA), NROWA)],
                    den_ref.at[cid, pl.ds(_m8(tid * NROWA), NROWA)])

    @pl.when(tid == 0)
    def _():
        pltpu.sync_copy(accd.at[pl.ds(TAIL0, TAIL)],
                        den_ref.at[cid, pl.ds(TAIL0, TAIL)])

    def head_body(h, _):
        _zero_slice(z64_ref, accm, tid)
        plsc.subcore_barrier()
        hsplat = jnp.full((16,), h, _i32)

        def sub_body(sub, _):
            base = _m8(wid * CHUNK + sub * SUB)
            rbase = _m8(wid * (CHUNK // IDXW) + sub * (SUB // IDXW))
            pltpu.sync_copy(gidx_ref.at[h, pl.ds(rbase, IDXR)], gv)
            pltpu.sync_copy(dst_ref.at[pl.ds(rbase, IDXR)], dstv)
            pltpu.sync_copy(ex_ref.at[pl.ds(base, SUB)], exv)

            def scale_scatter(j, half):
                off = half * IDXW

                def rowm(i, _):
                    vex = exv[j * IDXW + i, :].at[hsplat].get(
                        mode="promise_in_bounds")
                    for k in range(HID // 16):
                        s = pl.ds(k * 16, 16)
                        gxh[off + i, s] = gxh[off + i, s] * vex
                    return _

                lax.fori_loop(0, IDXW, rowm, None)
                pltpu.sync_copy(gxh.at[pl.ds(off, IDXW)],
                                accm.at[dstv.at[j]], add=True)

            def jpair(p, _):
                j0 = 2 * p
                j1 = 2 * p + 1
                c0 = pltpu.async_copy(tab_ref.at[gv.at[j0]],
                                      gxh.at[pl.ds(0, IDXW)], sem)
                c1 = pltpu.async_copy(tab_ref.at[gv.at[j1]],
                                      gxh.at[pl.ds(IDXW, IDXW)], sem2)
                c0.wait()
                scale_scatter(j0, 0)
                c1.wait()
                scale_scatter(j1, 1)
                return _

            lax.fori_loop(0, IDXR // 2, jpair, None)
            return _

        lax.fori_loop(0, NSUB, sub_body, None)
        plsc.subcore_barrier()
        pltpu.sync_copy(accm.at[pl.ds(_m8(tid * NROWA), NROWA)],
                        out_ref.at[cid, pl.ds(_m8(h * N + tid * NROWA), NROWA)])

        @pl.when(tid == 0)
        def _():
            pltpu.sync_copy(accm.at[pl.ds(TAIL0, TAIL)],
                            out_ref.at[cid, pl.ds(h * N + TAIL0, TAIL)])

        plsc.subcore_barrier()
        return _

    lax.fori_loop(0, nheads, head_body, None)


def _sc_agg(gidx3d, dst2d, ex, table, zeros16, zeros64, nheads):
    f = pl.kernel(
        functools.partial(_agg_body, nheads=nheads),
        out_type=[
            jax.ShapeDtypeStruct((NC, N, 16), _f32),
            jax.ShapeDtypeStruct((NC, nheads * N, HID), _f32),
        ],
        mesh=_MESH,
        compiler_params=_SCPARAMS,
        scratch_types=[
            pltpu.VMEM((IDXR, IDXW), _i32),
            pltpu.VMEM((IDXR, IDXW), _i32),
            pltpu.VMEM((2 * IDXW, HID), _f32),
            pltpu.VMEM((SUB, 16), _f32),
            pltpu.VMEM_SHARED((N, 16), _f32),
            pltpu.VMEM_SHARED((N, HID), _f32),
            pltpu.SemaphoreType.DMA,
            pltpu.SemaphoreType.DMA,
        ],
    )
    return f(gidx3d, dst2d, ex, table, zeros16, zeros64)


def _mlpg_body(src_ref, dstn_ref, ab_ref, c1_ref, r_ref,
               sv, dv, ga, gb, c1v, sem):
    cid, tid, wid = _worker()
    pltpu.sync_copy(c1_ref, c1v)

    for sub in range(NSUB):
        base = _m8(wid * CHUNK + sub * SUB)
        rbase = _m8(base // IDXW)
        pltpu.sync_copy(src_ref.at[pl.ds(rbase, IDXR)], sv)
        pltpu.sync_copy(dstn_ref.at[pl.ds(rbase, IDXR)], dv)
        for j in range(IDXR):
            pltpu.async_copy(ab_ref.at[sv.at[j]],
                             ga.at[pl.ds(j * IDXW, IDXW)], sem).wait()
        for half in range(2):
            for j2 in range(IDXR // 2):
                j = half * (IDXR // 2) + j2
                pltpu.async_copy(ab_ref.at[dv.at[j]],
                                 gb.at[pl.ds(j2 * IDXW, IDXW)], sem).wait()
            hbase = half * (SUB // 2)

            def row(i, _):
                for k in range(HID // 16):
                    s = pl.ds(k * 16, 16)
                    v = ga[hbase + i, s] + gb[i, s] + c1v[s]
                    ga[hbase + i, s] = jnp.maximum(v, 0.0)
                return _

            lax.fori_loop(0, SUB // 2, row, None)
        pltpu.sync_copy(ga, r_ref.at[pl.ds(base, SUB)])


def _sc_mlpg(src2d, dstn2d, ab_flat, c1):
    f = pl.kernel(
        _mlpg_body,
        out_type=jax.ShapeDtypeStruct((E, HID), _f32),
        mesh=_MESH,
        compiler_params=_SCPARAMS,
        scratch_types=[
            pltpu.VMEM((IDXR, IDXW), _i32),
            pltpu.VMEM((IDXR, IDXW), _i32),
            pltpu.VMEM((SUB, HID), _f32),
            pltpu.VMEM((SUB // 2, HID), _f32),
            pltpu.VMEM((HID,), _f32),
            pltpu.SemaphoreType.DMA,
        ],
    )
    return f(src2d, dstn2d, ab_flat, c1)


def kernel(x, edge_index, edge_attr, emb, W1, We1, as1, ad1, ae1, b1,
           W2, We2, as2, ad2, ae2, b2, M1, c1, M2, c2):
    src = edge_index[0]
    dst = edge_index[1]
    src2d = src.reshape(E // IDXW, IDXW)
    dst2d = dst.reshape(E // IDXW, IDXW)

    w1r = W1.reshape(D, HEADS, HID)
    W1sd = jnp.concatenate([
        jnp.einsum("dhc,hc->dh", w1r, as1),
        jnp.einsum("dhc,hc->dh", w1r, ad1)], axis=1)
    Wae = jnp.zeros((DE, 16), _f32)
    Wae = Wae.at[:, 0:HEADS].set(
        jnp.einsum("dhc,hc->dh", We1.reshape(DE, HEADS, HID), ae1))
    Wae = Wae.at[:, 8].set(We2 @ ae2[0])
    W2sd = jnp.zeros((HEADS * HID, 16), _f32)
    W2sd = W2sd.at[:, 0].set(W2 @ as2[0]).at[:, 1].set(W2 @ ad2[0])
    w1hm = w1r.transpose(1, 0, 2)
    w2hm = W2.reshape(HEADS, HID, HID)
    w2sdhm = W2sd.reshape(HEADS, HID, 16)
    b1hm = b1.reshape(HEADS, 1, HID)
    m2p = jnp.zeros((HID, 8), _f32).at[:, 0:3].set(M2)
    c2p = jnp.zeros((8,), _f32).at[0:3].set(c2)

    heads_off = (jnp.arange(HEADS, dtype=_i32) * N)[:, None, None]
    gidx1 = (src2d[None] + heads_off).astype(_i32)
    gidx2 = src2d[None]
    dstn2d = dst2d + N

    zeros16 = jnp.zeros((N, 16), _f32)
    zeros64 = jnp.zeros((N, HID), _f32)

    xh1hm, sd1 = _tc_node1(x.astype(_i32), emb, w1hm, W1sd)
    eatt = _tc_eatt(edge_attr, Wae)
    ex1 = _sc_att(src2d, dst2d, sd1, eatt, 1)
    den1, out1p = _sc_agg(gidx1, dst2d, ex1,
                          xh1hm.reshape(HEADS * N, HID), zeros16, zeros64,
                          HEADS)
    out1p = out1p.reshape(NC, HEADS, N, HID)

    xh2, sd2 = _tc_h1xh2(out1p, den1, b1hm, w2hm, w2sdhm)
    ex2 = _sc_att(src2d, dst2d, sd2, eatt, 2)
    den2, out2p = _sc_agg(gidx2, dst2d, ex2, xh2, zeros16, zeros64, 1)
    out2p = out2p.reshape(NC, 1, N, HID)

    ab = _tc_ab(out2p, den2, b2, M1[:HID], M1[HID:])
    r = _sc_mlpg(src2d, dstn2d, ab.reshape(2 * N, HID), c1)
    out = _tc_mlp(r, m2p, c2p)
    return out[:, 0:3]


# speedup vs baseline: 13.7816x; 1.0416x over previous
import functools

import jax
import jax.numpy as jnp
from jax import lax
from jax.experimental import pallas as pl
from jax.experimental.pallas import tpu as pltpu
from jax.experimental.pallas import tpu_sc as plsc

N = 10000
E = 160000
D = 128
HID = 64
HEADS = 8
DE = 16

NC = 2
NS = 16
NW = NC * NS
CHUNK = E // NW
IDXW = 125
IDXR = 8
SUB = IDXW * IDXR
NSUB = CHUNK // SUB
NROWA = 624
TAIL = N - NS * NROWA
TAIL0 = NS * NROWA

_f32 = jnp.float32
_i32 = jnp.int32


def _node1_body(x_ref, emb_ref, w1_ref, w1sd_ref, xh1_ref, sd1_ref):
    h = pl.program_id(1)
    ids = x_ref[...]
    iota = lax.broadcasted_iota(_i32, (ids.shape[0], 32), 1)
    onehot = (ids == iota).astype(_f32)
    h0 = jnp.dot(onehot, emb_ref[...], preferred_element_type=_f32)
    xh1_ref[0] = jnp.dot(h0, w1_ref[0], preferred_element_type=_f32)

    @pl.when(h == 0)
    def _():
        sd1_ref[...] = jnp.dot(h0, w1sd_ref[...], preferred_element_type=_f32)


def _tc_node1(x, emb, w1hm, w1sd):
    bn = 1000
    return pl.pallas_call(
        _node1_body,
        grid=(N // bn, HEADS),
        in_specs=[
            pl.BlockSpec((bn, 1), lambda nt, h: (nt, 0)),
            pl.BlockSpec((32, D), lambda nt, h: (0, 0)),
            pl.BlockSpec((1, D, HID), lambda nt, h: (h, 0, 0)),
            pl.BlockSpec((D, 16), lambda nt, h: (0, 0)),
        ],
        out_specs=[
            pl.BlockSpec((1, bn, HID), lambda nt, h: (h, nt, 0)),
            pl.BlockSpec((bn, 16), lambda nt, h: (nt, 0)),
        ],
        out_shape=[
            jax.ShapeDtypeStruct((HEADS, N, HID), _f32),
            jax.ShapeDtypeStruct((N, 16), _f32),
        ],
    )(x, emb, w1hm, w1sd)


def _eatt_body(ea_ref, wae_ref, out_ref):
    out_ref[...] = jnp.dot(ea_ref[...], wae_ref[...],
                           preferred_element_type=_f32)


def _tc_eatt(ea, wae):
    be = 2000
    return pl.pallas_call(
        _eatt_body,
        grid=(E // be,),
        in_specs=[
            pl.BlockSpec((be, DE), lambda i: (i, 0)),
            pl.BlockSpec((DE, 16), lambda i: (0, 0)),
        ],
        out_specs=pl.BlockSpec((be, 16), lambda i: (i, 0)),
        out_shape=jax.ShapeDtypeStruct((E, 16), _f32),
    )(ea, wae)


def _h1xh2_body(p0_ref, p1_ref, d0_ref, d1_ref, b1_ref, w2_ref, w2sd_ref,
                xh2_ref, sd2_ref):
    h = pl.program_id(1)
    den = d0_ref[0] + d1_ref[0] + 1e-16
    lane = lax.broadcasted_iota(_i32, den.shape, 1)
    dh = jnp.sum(jnp.where(lane == h, den, 0.0), axis=1, keepdims=True)
    hb = (p0_ref[0, 0] + p1_ref[0, 0]) / dh + b1_ref[0]
    hb = jnp.where(hb > 0, hb, jnp.exp(hb) - 1.0)
    dx = jnp.dot(hb, w2_ref[0], preferred_element_type=_f32)
    ds = jnp.dot(hb, w2sd_ref[0], preferred_element_type=_f32)

    @pl.when(h == 0)
    def _():
        xh2_ref[...] = jnp.zeros_like(xh2_ref)
        sd2_ref[...] = jnp.zeros_like(sd2_ref)

    xh2_ref[...] += dx
    sd2_ref[...] += ds


def _tc_h1xh2(out1p, den1, b1hm, w2hm, w2sdhm):
    bn = 1000
    return pl.pallas_call(
        _h1xh2_body,
        grid=(N // bn, HEADS),
        in_specs=[
            pl.BlockSpec((1, 1, bn, HID), lambda nt, h: (0, h, nt, 0)),
            pl.BlockSpec((1, 1, bn, HID), lambda nt, h: (1, h, nt, 0)),
            pl.BlockSpec((1, bn, 16), lambda nt, h: (0, nt, 0)),
            pl.BlockSpec((1, bn, 16), lambda nt, h: (1, nt, 0)),
            pl.BlockSpec((1, 1, HID), lambda nt, h: (h, 0, 0)),
            pl.BlockSpec((1, HID, HID), lambda nt, h: (h, 0, 0)),
            pl.BlockSpec((1, HID, 16), lambda nt, h: (h, 0, 0)),
        ],
        out_specs=[
            pl.BlockSpec((bn, HID), lambda nt, h: (nt, 0)),
            pl.BlockSpec((bn, 16), lambda nt, h: (nt, 0)),
        ],
        out_shape=[
            jax.ShapeDtypeStruct((N, HID), _f32),
            jax.ShapeDtypeStruct((N, 16), _f32),
        ],
    )(out1p, out1p, den1, den1, b1hm, w2hm, w2sdhm)


def _ab_body(p0_ref, p1_ref, d0_ref, d1_ref, b2_ref, m1a_ref, m1b_ref,
             ab_ref):
    den = d0_ref[0] + d1_ref[0] + 1e-16
    h2 = (p0_ref[0, 0] + p1_ref[0, 0]) / den[:, 0:1] + b2_ref[...]
    h2 = jnp.where(h2 > 0, h2, jnp.exp(h2) - 1.0)
    ab_ref[0] = jnp.dot(h2, m1a_ref[...], preferred_element_type=_f32)
    ab_ref[1] = jnp.dot(h2, m1b_ref[...], preferred_element_type=_f32)


def _tc_ab(out2p, den2, b2, m1a, m1b):
    bn = 1000
    return pl.pallas_call(
        _ab_body,
        grid=(N // bn,),
        in_specs=[
            pl.BlockSpec((1, 1, bn, HID), lambda nt: (0, 0, nt, 0)),
            pl.BlockSpec((1, 1, bn, HID), lambda nt: (1, 0, nt, 0)),
            pl.BlockSpec((1, bn, 16), lambda nt: (0, nt, 0)),
            pl.BlockSpec((1, bn, 16), lambda nt: (1, nt, 0)),
            pl.BlockSpec((1, HID), lambda nt: (0, 0)),
            pl.BlockSpec((HID, HID), lambda nt: (0, 0)),
            pl.BlockSpec((HID, HID), lambda nt: (0, 0)),
        ],
        out_specs=pl.BlockSpec((2, bn, HID), lambda nt: (0, nt, 0)),
        out_shape=jax.ShapeDtypeStruct((2, N, HID), _f32),
    )(out2p, out2p, den2, den2, b2.reshape(1, HID), m1a, m1b)


def _mlp_body(r_ref, m2_ref, c2_ref, out_ref):
    out_ref[...] = jnp.dot(r_ref[...], m2_ref[...],
                           preferred_element_type=_f32) + c2_ref[...]


def _tc_mlp(r, m2p, c2p):
    be = 2000
    return pl.pallas_call(
        _mlp_body,
        grid=(E // be,),
        in_specs=[
            pl.BlockSpec((be, HID), lambda i: (i, 0)),
            pl.BlockSpec((HID, 8), lambda i: (0, 0)),
            pl.BlockSpec((1, 8), lambda i: (0, 0)),
        ],
        out_specs=pl.BlockSpec((be, 8), lambda i: (i, 0)),
        out_shape=jax.ShapeDtypeStruct((E, 8), _f32),
    )(r, m2p, c2p.reshape(1, 8))


_MESH = plsc.VectorSubcoreMesh(core_axis_name="c", subcore_axis_name="s")
_SCPARAMS = pltpu.CompilerParams(use_tc_tiling_on_sc=False)


def _worker():
    cid = lax.axis_index("c")
    tid = lax.axis_index("s")
    return cid, tid, cid * NS + tid


def _m8(v):
    return pl.multiple_of(v, 8)


def _att_body(src_ref, dst_ref, sd_ref, eatt_ref, ex_ref,
              srcv, dstv, gsv, gdv, gev, exv, sem, sem2, *, layer):
    cid, tid, wid = _worker()
    lane = lax.iota(_i32, 16)
    if layer == 1:
        perm_s = None
        perm_d = jnp.minimum(lane + 8, 15)
        perm_e = None
        nlanes = 8
    else:
        perm_s = lane * 0
        perm_d = lane * 0 + 1
        perm_e = lane * 0 + 8
        nlanes = 1

    def _take(vec, perm):
        if perm is None:
            return vec
        return vec.at[perm].get(mode="promise_in_bounds")

    for sub in range(NSUB):
        base = _m8(wid * CHUNK + sub * SUB)
        rbase = _m8(base // IDXW)
        pltpu.sync_copy(src_ref.at[pl.ds(rbase, IDXR)], srcv)
        pltpu.sync_copy(dst_ref.at[pl.ds(rbase, IDXR)], dstv)
        cps = []
        for j in range(IDXR):
            cps.append(pltpu.async_copy(sd_ref.at[srcv.at[j]],
                                        gsv.at[pl.ds(j * IDXW, IDXW)], sem))
            cps.append(pltpu.async_copy(sd_ref.at[dstv.at[j]],
                                        gdv.at[pl.ds(j * IDXW, IDXW)], sem2))
        for c in cps:
            c.wait()
        pltpu.sync_copy(eatt_ref.at[pl.ds(base, SUB)], gev)

        def row(i, _):
            vs = _take(gsv[i, :], perm_s)
            vd = _take(gdv[i, :], perm_d)
            ve = _take(gev[i, :], perm_e)
            a = vs + vd + ve
            a = jnp.where(a > 0, a, 0.2 * a)
            e = jnp.exp(a)
            exv[i, :] = jnp.where(lane < nlanes, e, 0.0)
            return _

        lax.fori_loop(0, SUB, row, None)
        pltpu.sync_copy(exv, ex_ref.at[pl.ds(base, SUB)])


def _sc_att(src2d, dst2d, sd, eatt, layer):
    f = pl.kernel(
        functools.partial(_att_body, layer=layer),
        out_type=jax.ShapeDtypeStruct((E, 16), _f32),
        mesh=_MESH,
        compiler_params=_SCPARAMS,
        scratch_types=[
            pltpu.VMEM((IDXR, IDXW), _i32),
            pltpu.VMEM((IDXR, IDXW), _i32),
            pltpu.VMEM((SUB, 16), _f32),
            pltpu.VMEM((SUB, 16), _f32),
            pltpu.VMEM((SUB, 16), _f32),
            pltpu.VMEM((SUB, 16), _f32),
            pltpu.SemaphoreType.DMA,
            pltpu.SemaphoreType.DMA,
        ],
    )
    return f(src2d, dst2d, sd, eatt)


def _zero_slice(zeros_ref, acc, tid):
    pltpu.sync_copy(zeros_ref.at[pl.ds(_m8(tid * NROWA), NROWA)],
                    acc.at[pl.ds(_m8(tid * NROWA), NROWA)])

    @pl.when(tid == 0)
    def _():
        pltpu.sync_copy(zeros_ref.at[pl.ds(TAIL0, TAIL)],
                        acc.at[pl.ds(TAIL0, TAIL)])


def _agg_body(gidx_ref, dst_ref, ex_ref, tab_ref, z16_ref, z64_ref,
              den_ref, out_ref,
              gv, dstv, gxh, exv, accd, accm, sem, sem2, *, nheads):
    cid, tid, wid = _worker()

    _zero_slice(z16_ref, accd, tid)
    plsc.subcore_barrier()
    for sub in range(NSUB):
        base = _m8(wid * CHUNK + sub * SUB)
        rbase = _m8(base // IDXW)
        pltpu.sync_copy(dst_ref.at[pl.ds(rbase, IDXR)], dstv)
        pltpu.sync_copy(ex_ref.at[pl.ds(base, SUB)], exv)
        for j in range(IDXR):
            pltpu.sync_copy(exv.at[pl.ds(j * IDXW, IDXW)],
                            accd.at[dstv.at[j]], add=True)
    plsc.subcore_barrier()
    pltpu.sync_copy(accd.at[pl.ds(_m8(tid * NROWA), NROWA)],
                    den_ref.at[cid, pl.ds(_m8(tid * NROWA), NROWA)])

    @pl.when(tid == 0)
    def _():
        pltpu.sync_copy(accd.at[pl.ds(TAIL0, TAIL)],
                        den_ref.at[cid, pl.ds(TAIL0, TAIL)])

    def head_body(h, _):
        _zero_slice(z64_ref, accm, tid)
        plsc.subcore_barrier()
        hsplat = jnp.full((16,), h, _i32)

        def sub_body(sub, _):
            base = _m8(wid * CHUNK + sub * SUB)
            rbase = _m8(wid * (CHUNK // IDXW) + sub * (SUB // IDXW))
            pltpu.sync_copy(gidx_ref.at[h, pl.ds(rbase, IDXR)], gv)
            pltpu.sync_copy(dst_ref.at[pl.ds(rbase, IDXR)], dstv)
            pltpu.sync_copy(ex_ref.at[pl.ds(base, SUB)], exv)

            def scale_scatter(j, half):
                off = half * IDXW

                def rowm(i, _):
                    vex = exv[j * IDXW + i, :].at[hsplat].get(
                        mode="promise_in_bounds")
                    for k in range(HID // 16):
                        s = pl.ds(k * 16, 16)
                        gxh[off + i, s] = gxh[off + i, s] * vex
                    return _

                lax.fori_loop(0, IDXW, rowm, None)
                pltpu.sync_copy(gxh.at[pl.ds(off, IDXW)],
                                accm.at[dstv.at[j]], add=True)

            def jpair(p, _):
                j0 = 2 * p
                j1 = 2 * p + 1
                c0 = pltpu.async_copy(tab_ref.at[gv.at[j0]],
                                      gxh.at[pl.ds(0, IDXW)], sem)
                c1 = pltpu.async_copy(tab_ref.at[gv.at[j1]],
                                      gxh.at[pl.ds(IDXW, IDXW)], sem2)
                c0.wait()
                scale_scatter(j0, 0)
                c1.wait()
                scale_scatter(j1, 1)
                return _

            lax.fori_loop(0, IDXR // 2, jpair, None)
            return _

        lax.fori_loop(0, NSUB, sub_body, None)
        plsc.subcore_barrier()
        pltpu.sync_copy(accm.at[pl.ds(_m8(tid * NROWA), NROWA)],
                        out_ref.at[cid, pl.ds(_m8(h * N + tid * NROWA), NROWA)])

        @pl.when(tid == 0)
        def _():
            pltpu.sync_copy(accm.at[pl.ds(TAIL0, TAIL)],
                            out_ref.at[cid, pl.ds(h * N + TAIL0, TAIL)])

        plsc.subcore_barrier()
        return _

    lax.fori_loop(0, nheads, head_body, None)


def _sc_agg(gidx3d, dst2d, ex, table, zeros16, zeros64, nheads):
    f = pl.kernel(
        functools.partial(_agg_body, nheads=nheads),
        out_type=[
            jax.ShapeDtypeStruct((NC, N, 16), _f32),
            jax.ShapeDtypeStruct((NC, nheads * N, HID), _f32),
        ],
        mesh=_MESH,
        compiler_params=_SCPARAMS,
        scratch_types=[
            pltpu.VMEM((IDXR, IDXW), _i32),
            pltpu.VMEM((IDXR, IDXW), _i32),
            pltpu.VMEM((2 * IDXW, HID), _f32),
            pltpu.VMEM((SUB, 16), _f32),
            pltpu.VMEM_SHARED((N, 16), _f32),
            pltpu.VMEM_SHARED((N, HID), _f32),
            pltpu.SemaphoreType.DMA,
            pltpu.SemaphoreType.DMA,
        ],
    )
    return f(gidx3d, dst2d, ex, table, zeros16, zeros64)


def _mlpg_body(src_ref, dstn_ref, ab_ref, c1_ref, r_ref,
               sv, dv, ga, gb, c1v, sem, sem2):
    cid, tid, wid = _worker()
    pltpu.sync_copy(c1_ref, c1v)

    for sub in range(NSUB):
        base = _m8(wid * CHUNK + sub * SUB)
        rbase = _m8(base // IDXW)
        pltpu.sync_copy(src_ref.at[pl.ds(rbase, IDXR)], sv)
        pltpu.sync_copy(dstn_ref.at[pl.ds(rbase, IDXR)], dv)
        cga = [pltpu.async_copy(ab_ref.at[sv.at[j]],
                                ga.at[pl.ds(j * IDXW, IDXW)], sem)
               for j in range(IDXR)]
        for half in range(2):
            cgb = [pltpu.async_copy(
                       ab_ref.at[dv.at[half * (IDXR // 2) + j2]],
                       gb.at[pl.ds(j2 * IDXW, IDXW)], sem2)
                   for j2 in range(IDXR // 2)]
            if half == 0:
                for c in cga:
                    c.wait()
            for c in cgb:
                c.wait()
            hbase = half * (SUB // 2)

            def row(i, _):
                for k in range(HID // 16):
                    s = pl.ds(k * 16, 16)
                    v = ga[hbase + i, s] + gb[i, s] + c1v[s]
                    ga[hbase + i, s] = jnp.maximum(v, 0.0)
                return _

            lax.fori_loop(0, SUB // 2, row, None)
        pltpu.sync_copy(ga, r_ref.at[pl.ds(base, SUB)])


def _sc_mlpg(src2d, dstn2d, ab_flat, c1):
    f = pl.kernel(
        _mlpg_body,
        out_type=jax.ShapeDtypeStruct((E, HID), _f32),
        mesh=_MESH,
        compiler_params=_SCPARAMS,
        scratch_types=[
            pltpu.VMEM((IDXR, IDXW), _i32),
            pltpu.VMEM((IDXR, IDXW), _i32),
            pltpu.VMEM((SUB, HID), _f32),
            pltpu.VMEM((SUB // 2, HID), _f32),
            pltpu.VMEM((HID,), _f32),
            pltpu.SemaphoreType.DMA,
            pltpu.SemaphoreType.DMA,
        ],
    )
    return f(src2d, dstn2d, ab_flat, c1)


def kernel(x, edge_index, edge_attr, emb, W1, We1, as1, ad1, ae1, b1,
           W2, We2, as2, ad2, ae2, b2, M1, c1, M2, c2):
    src = edge_index[0]
    dst = edge_index[1]
    src2d = src.reshape(E // IDXW, IDXW)
    dst2d = dst.reshape(E // IDXW, IDXW)

    w1r = W1.reshape(D, HEADS, HID)
    W1sd = jnp.concatenate([
        jnp.einsum("dhc,hc->dh", w1r, as1),
        jnp.einsum("dhc,hc->dh", w1r, ad1)], axis=1)
    Wae = jnp.zeros((DE, 16), _f32)
    Wae = Wae.at[:, 0:HEADS].set(
        jnp.einsum("dhc,hc->dh", We1.reshape(DE, HEADS, HID), ae1))
    Wae = Wae.at[:, 8].set(We2 @ ae2[0])
    W2sd = jnp.zeros((HEADS * HID, 16), _f32)
    W2sd = W2sd.at[:, 0].set(W2 @ as2[0]).at[:, 1].set(W2 @ ad2[0])
    w1hm = w1r.transpose(1, 0, 2)
    w2hm = W2.reshape(HEADS, HID, HID)
    w2sdhm = W2sd.reshape(HEADS, HID, 16)
    b1hm = b1.reshape(HEADS, 1, HID)
    m2p = jnp.zeros((HID, 8), _f32).at[:, 0:3].set(M2)
    c2p = jnp.zeros((8,), _f32).at[0:3].set(c2)

    heads_off = (jnp.arange(HEADS, dtype=_i32) * N)[:, None, None]
    gidx1 = (src2d[None] + heads_off).astype(_i32)
    gidx2 = src2d[None]
    dstn2d = dst2d + N

    zeros16 = jnp.zeros((N, 16), _f32)
    zeros64 = jnp.zeros((N, HID), _f32)

    xh1hm, sd1 = _tc_node1(x.astype(_i32), emb, w1hm, W1sd)
    eatt = _tc_eatt(edge_attr, Wae)
    ex1 = _sc_att(src2d, dst2d, sd1, eatt, 1)
    den1, out1p = _sc_agg(gidx1, dst2d, ex1,
                          xh1hm.reshape(HEADS * N, HID), zeros16, zeros64,
                          HEADS)
    out1p = out1p.reshape(NC, HEADS, N, HID)

    xh2, sd2 = _tc_h1xh2(out1p, den1, b1hm, w2hm, w2sdhm)
    ex2 = _sc_att(src2d, dst2d, sd2, eatt, 2)
    den2, out2p = _sc_agg(gidx2, dst2d, ex2, xh2, zeros16, zeros64, 1)
    out2p = out2p.reshape(NC, 1, N, HID)

    ab = _tc_ab(out2p, den2, b2, M1[:HID], M1[HID:])
    r = _sc_mlpg(src2d, dstn2d, ab.reshape(2 * N, HID), c1)
    out = _tc_mlp(r, m2p, c2p)
    return out[:, 0:3]


# speedup vs baseline: 14.0322x; 1.0182x over previous
import functools

import jax
import jax.numpy as jnp
from jax import lax
from jax.experimental import pallas as pl
from jax.experimental.pallas import tpu as pltpu
from jax.experimental.pallas import tpu_sc as plsc

N = 10000
E = 160000
D = 128
HID = 64
HEADS = 8
DE = 16

NC = 2
NS = 16
NW = NC * NS
CHUNK = E // NW
IDXW = 125
IDXR = 8
SUB = IDXW * IDXR
NSUB = CHUNK // SUB
NROWA = 624
TAIL = N - NS * NROWA
TAIL0 = NS * NROWA

_f32 = jnp.float32
_i32 = jnp.int32


def _node1_body(x_ref, emb_ref, w1_ref, w1sd_ref, xh1_ref, sd1_ref):
    h = pl.program_id(1)
    ids = x_ref[...]
    iota = lax.broadcasted_iota(_i32, (ids.shape[0], 32), 1)
    onehot = (ids == iota).astype(_f32)
    h0 = jnp.dot(onehot, emb_ref[...], preferred_element_type=_f32)
    xh1_ref[0] = jnp.dot(h0, w1_ref[0], preferred_element_type=_f32)

    @pl.when(h == 0)
    def _():
        sd1_ref[...] = jnp.dot(h0, w1sd_ref[...], preferred_element_type=_f32)


def _tc_node1(x, emb, w1hm, w1sd):
    bn = 1000
    return pl.pallas_call(
        _node1_body,
        grid=(N // bn, HEADS),
        in_specs=[
            pl.BlockSpec((bn, 1), lambda nt, h: (nt, 0)),
            pl.BlockSpec((32, D), lambda nt, h: (0, 0)),
            pl.BlockSpec((1, D, HID), lambda nt, h: (h, 0, 0)),
            pl.BlockSpec((D, 16), lambda nt, h: (0, 0)),
        ],
        out_specs=[
            pl.BlockSpec((1, bn, HID), lambda nt, h: (h, nt, 0)),
            pl.BlockSpec((bn, 16), lambda nt, h: (nt, 0)),
        ],
        out_shape=[
            jax.ShapeDtypeStruct((HEADS, N, HID), _f32),
            jax.ShapeDtypeStruct((N, 16), _f32),
        ],
    )(x, emb, w1hm, w1sd)


def _eatt_body(ea_ref, wae_ref, out_ref):
    out_ref[...] = jnp.dot(ea_ref[...], wae_ref[...],
                           preferred_element_type=_f32)


def _tc_eatt(ea, wae):
    be = 2000
    return pl.pallas_call(
        _eatt_body,
        grid=(E // be,),
        in_specs=[
            pl.BlockSpec((be, DE), lambda i: (i, 0)),
            pl.BlockSpec((DE, 16), lambda i: (0, 0)),
        ],
        out_specs=pl.BlockSpec((be, 16), lambda i: (i, 0)),
        out_shape=jax.ShapeDtypeStruct((E, 16), _f32),
    )(ea, wae)


def _h1xh2_body(p0_ref, p1_ref, d0_ref, d1_ref, b1_ref, w2_ref, w2sd_ref,
                xh2_ref, sd2_ref):
    h = pl.program_id(1)
    den = d0_ref[0] + d1_ref[0] + 1e-16
    lane = lax.broadcasted_iota(_i32, den.shape, 1)
    dh = jnp.sum(jnp.where(lane == h, den, 0.0), axis=1, keepdims=True)
    hb = (p0_ref[0, 0] + p1_ref[0, 0]) / dh + b1_ref[0]
    hb = jnp.where(hb > 0, hb, jnp.exp(hb) - 1.0)
    dx = jnp.dot(hb, w2_ref[0], preferred_element_type=_f32)
    ds = jnp.dot(hb, w2sd_ref[0], preferred_element_type=_f32)

    @pl.when(h == 0)
    def _():
        xh2_ref[...] = jnp.zeros_like(xh2_ref)
        sd2_ref[...] = jnp.zeros_like(sd2_ref)

    xh2_ref[...] += dx
    sd2_ref[...] += ds


def _tc_h1xh2(out1p, den1, b1hm, w2hm, w2sdhm):
    bn = 1000
    return pl.pallas_call(
        _h1xh2_body,
        grid=(N // bn, HEADS),
        in_specs=[
            pl.BlockSpec((1, 1, bn, HID), lambda nt, h: (0, h, nt, 0)),
            pl.BlockSpec((1, 1, bn, HID), lambda nt, h: (1, h, nt, 0)),
            pl.BlockSpec((1, bn, 16), lambda nt, h: (0, nt, 0)),
            pl.BlockSpec((1, bn, 16), lambda nt, h: (1, nt, 0)),
            pl.BlockSpec((1, 1, HID), lambda nt, h: (h, 0, 0)),
            pl.BlockSpec((1, HID, HID), lambda nt, h: (h, 0, 0)),
            pl.BlockSpec((1, HID, 16), lambda nt, h: (h, 0, 0)),
        ],
        out_specs=[
            pl.BlockSpec((bn, HID), lambda nt, h: (nt, 0)),
            pl.BlockSpec((bn, 16), lambda nt, h: (nt, 0)),
        ],
        out_shape=[
            jax.ShapeDtypeStruct((N, HID), _f32),
            jax.ShapeDtypeStruct((N, 16), _f32),
        ],
    )(out1p, out1p, den1, den1, b1hm, w2hm, w2sdhm)


def _ab_body(p0_ref, p1_ref, d0_ref, d1_ref, b2_ref, m1a_ref, m1b_ref,
             ab_ref):
    den = d0_ref[0] + d1_ref[0] + 1e-16
    h2 = (p0_ref[0, 0] + p1_ref[0, 0]) / den[:, 0:1] + b2_ref[...]
    h2 = jnp.where(h2 > 0, h2, jnp.exp(h2) - 1.0)
    ab_ref[0] = jnp.dot(h2, m1a_ref[...], preferred_element_type=_f32)
    ab_ref[1] = jnp.dot(h2, m1b_ref[...], preferred_element_type=_f32)


def _tc_ab(out2p, den2, b2, m1a, m1b):
    bn = 1000
    return pl.pallas_call(
        _ab_body,
        grid=(N // bn,),
        in_specs=[
            pl.BlockSpec((1, 1, bn, HID), lambda nt: (0, 0, nt, 0)),
            pl.BlockSpec((1, 1, bn, HID), lambda nt: (1, 0, nt, 0)),
            pl.BlockSpec((1, bn, 16), lambda nt: (0, nt, 0)),
            pl.BlockSpec((1, bn, 16), lambda nt: (1, nt, 0)),
            pl.BlockSpec((1, HID), lambda nt: (0, 0)),
            pl.BlockSpec((HID, HID), lambda nt: (0, 0)),
            pl.BlockSpec((HID, HID), lambda nt: (0, 0)),
        ],
        out_specs=pl.BlockSpec((2, bn, HID), lambda nt: (0, nt, 0)),
        out_shape=jax.ShapeDtypeStruct((2, N, HID), _f32),
    )(out2p, out2p, den2, den2, b2.reshape(1, HID), m1a, m1b)


def _mlp_body(r_ref, m2_ref, c2_ref, out_ref):
    out_ref[...] = jnp.dot(r_ref[...], m2_ref[...],
                           preferred_element_type=_f32) + c2_ref[...]


def _tc_mlp(r, m2p, c2p):
    be = 2000
    return pl.pallas_call(
        _mlp_body,
        grid=(E // be,),
        in_specs=[
            pl.BlockSpec((be, HID), lambda i: (i, 0)),
            pl.BlockSpec((HID, 8), lambda i: (0, 0)),
            pl.BlockSpec((1, 8), lambda i: (0, 0)),
        ],
        out_specs=pl.BlockSpec((be, 8), lambda i: (i, 0)),
        out_shape=jax.ShapeDtypeStruct((E, 8), _f32),
    )(r, m2p, c2p.reshape(1, 8))


_MESH = plsc.VectorSubcoreMesh(core_axis_name="c", subcore_axis_name="s")
_SCPARAMS = pltpu.CompilerParams(use_tc_tiling_on_sc=False)


def _worker():
    cid = lax.axis_index("c")
    tid = lax.axis_index("s")
    return cid, tid, cid * NS + tid


def _m8(v):
    return pl.multiple_of(v, 8)


def _att_body(src_ref, dst_ref, sd_ref, eatt_ref, ex_ref,
              srcv, dstv, gsv, gdv, gev, exv, sem, sem2, *, layer):
    cid, tid, wid = _worker()
    lane = lax.iota(_i32, 16)
    if layer == 1:
        perm_s = None
        perm_d = jnp.minimum(lane + 8, 15)
        perm_e = None
        nlanes = 8
    else:
        perm_s = lane * 0
        perm_d = lane * 0 + 1
        perm_e = lane * 0 + 8
        nlanes = 1

    def _take(vec, perm):
        if perm is None:
            return vec
        return vec.at[perm].get(mode="promise_in_bounds")

    for sub in range(NSUB):
        base = _m8(wid * CHUNK + sub * SUB)
        rbase = _m8(base // IDXW)
        pltpu.sync_copy(src_ref.at[pl.ds(rbase, IDXR)], srcv)
        pltpu.sync_copy(dst_ref.at[pl.ds(rbase, IDXR)], dstv)
        cps = []
        for j in range(IDXR):
            cps.append(pltpu.async_copy(sd_ref.at[srcv.at[j]],
                                        gsv.at[pl.ds(j * IDXW, IDXW)], sem))
            cps.append(pltpu.async_copy(sd_ref.at[dstv.at[j]],
                                        gdv.at[pl.ds(j * IDXW, IDXW)], sem2))
        for c in cps:
            c.wait()
        pltpu.sync_copy(eatt_ref.at[pl.ds(base, SUB)], gev)

        def row(i, _):
            vs = _take(gsv[i, :], perm_s)
            vd = _take(gdv[i, :], perm_d)
            ve = _take(gev[i, :], perm_e)
            a = vs + vd + ve
            a = jnp.where(a > 0, a, 0.2 * a)
            e = jnp.exp(a)
            exv[i, :] = jnp.where(lane < nlanes, e, 0.0)
            return _

        lax.fori_loop(0, SUB, row, None)
        pltpu.sync_copy(exv, ex_ref.at[pl.ds(base, SUB)])


def _sc_att(src2d, dst2d, sd, eatt, layer):
    f = pl.kernel(
        functools.partial(_att_body, layer=layer),
        out_type=jax.ShapeDtypeStruct((E, 16), _f32),
        mesh=_MESH,
        compiler_params=_SCPARAMS,
        scratch_types=[
            pltpu.VMEM((IDXR, IDXW), _i32),
            pltpu.VMEM((IDXR, IDXW), _i32),
            pltpu.VMEM((SUB, 16), _f32),
            pltpu.VMEM((SUB, 16), _f32),
            pltpu.VMEM((SUB, 16), _f32),
            pltpu.VMEM((SUB, 16), _f32),
            pltpu.SemaphoreType.DMA,
            pltpu.SemaphoreType.DMA,
        ],
    )
    return f(src2d, dst2d, sd, eatt)


def _zero_slice(zeros_ref, acc, tid):
    pltpu.sync_copy(zeros_ref.at[pl.ds(_m8(tid * NROWA), NROWA)],
                    acc.at[pl.ds(_m8(tid * NROWA), NROWA)])

    @pl.when(tid == 0)
    def _():
        pltpu.sync_copy(zeros_ref.at[pl.ds(TAIL0, TAIL)],
                        acc.at[pl.ds(TAIL0, TAIL)])


def _agg_body(gidx_ref, dst_ref, ex_ref, tab_ref, z16_ref, z64_ref,
              den_ref, out_ref,
              gv, dstv, gxh, exv, accd, accm, s0, s1, s2, s3, *, nheads):
    cid, tid, wid = _worker()

    _zero_slice(z16_ref, accd, tid)
    plsc.subcore_barrier()
    for sub in range(NSUB):
        base = _m8(wid * CHUNK + sub * SUB)
        rbase = _m8(base // IDXW)
        pltpu.sync_copy(dst_ref.at[pl.ds(rbase, IDXR)], dstv)
        pltpu.sync_copy(ex_ref.at[pl.ds(base, SUB)], exv)
        for j in range(IDXR):
            pltpu.sync_copy(exv.at[pl.ds(j * IDXW, IDXW)],
                            accd.at[dstv.at[j]], add=True)
    plsc.subcore_barrier()
    pltpu.sync_copy(accd.at[pl.ds(_m8(tid * NROWA), NROWA)],
                    den_ref.at[cid, pl.ds(_m8(tid * NROWA), NROWA)])

    @pl.when(tid == 0)
    def _():
        pltpu.sync_copy(accd.at[pl.ds(TAIL0, TAIL)],
                        den_ref.at[cid, pl.ds(TAIL0, TAIL)])

    def head_body(h, _):
        _zero_slice(z64_ref, accm, tid)
        plsc.subcore_barrier()
        hsplat = jnp.full((16,), h, _i32)

        def sub_body(sub, _):
            base = _m8(wid * CHUNK + sub * SUB)
            rbase = _m8(wid * (CHUNK // IDXW) + sub * (SUB // IDXW))
            pltpu.sync_copy(gidx_ref.at[h, pl.ds(rbase, IDXR)], gv)
            pltpu.sync_copy(dst_ref.at[pl.ds(rbase, IDXR)], dstv)
            pltpu.sync_copy(ex_ref.at[pl.ds(base, SUB)], exv)

            def scale_scatter(j, half):
                off = half * IDXW

                def rowm(i, _):
                    vex = exv[j * IDXW + i, :].at[hsplat].get(
                        mode="promise_in_bounds")
                    for k in range(HID // 16):
                        s = pl.ds(k * 16, 16)
                        gxh[off + i, s] = gxh[off + i, s] * vex
                    return _

                lax.fori_loop(0, IDXW, rowm, None)
                pltpu.sync_copy(gxh.at[pl.ds(off, IDXW)],
                                accm.at[dstv.at[j]], add=True)

            def jquad(p, _):
                jb = 4 * p
                sems = (s0, s1, s2, s3)
                cps = [pltpu.async_copy(tab_ref.at[gv.at[jb + q]],
                                        gxh.at[pl.ds(q * IDXW, IDXW)],
                                        sems[q])
                       for q in range(4)]
                for q in range(4):
                    cps[q].wait()
                    scale_scatter(jb + q, q)
                return _

            lax.fori_loop(0, IDXR // 4, jquad, None)
            return _

        lax.fori_loop(0, NSUB, sub_body, None)
        plsc.subcore_barrier()
        pltpu.sync_copy(accm.at[pl.ds(_m8(tid * NROWA), NROWA)],
                        out_ref.at[cid, pl.ds(_m8(h * N + tid * NROWA), NROWA)])

        @pl.when(tid == 0)
        def _():
            pltpu.sync_copy(accm.at[pl.ds(TAIL0, TAIL)],
                            out_ref.at[cid, pl.ds(h * N + TAIL0, TAIL)])

        plsc.subcore_barrier()
        return _

    lax.fori_loop(0, nheads, head_body, None)


def _sc_agg(gidx3d, dst2d, ex, table, zeros16, zeros64, nheads):
    f = pl.kernel(
        functools.partial(_agg_body, nheads=nheads),
        out_type=[
            jax.ShapeDtypeStruct((NC, N, 16), _f32),
            jax.ShapeDtypeStruct((NC, nheads * N, HID), _f32),
        ],
        mesh=_MESH,
        compiler_params=_SCPARAMS,
        scratch_types=[
            pltpu.VMEM((IDXR, IDXW), _i32),
            pltpu.VMEM((IDXR, IDXW), _i32),
            pltpu.VMEM((4 * IDXW, HID), _f32),
            pltpu.VMEM((SUB, 16), _f32),
            pltpu.VMEM_SHARED((N, 16), _f32),
            pltpu.VMEM_SHARED((N, HID), _f32),
            pltpu.SemaphoreType.DMA,
            pltpu.SemaphoreType.DMA,
            pltpu.SemaphoreType.DMA,
            pltpu.SemaphoreType.DMA,
        ],
    )
    return f(gidx3d, dst2d, ex, table, zeros16, zeros64)


def _mlpg_body(src_ref, dstn_ref, ab_ref, c1_ref, r_ref,
               sv, dv, ga, gb, c1v, sem, sem2):
    cid, tid, wid = _worker()
    pltpu.sync_copy(c1_ref, c1v)

    for sub in range(NSUB):
        base = _m8(wid * CHUNK + sub * SUB)
        rbase = _m8(base // IDXW)
        pltpu.sync_copy(src_ref.at[pl.ds(rbase, IDXR)], sv)
        pltpu.sync_copy(dstn_ref.at[pl.ds(rbase, IDXR)], dv)
        cga = [pltpu.async_copy(ab_ref.at[sv.at[j]],
                                ga.at[pl.ds(j * IDXW, IDXW)], sem)
               for j in range(IDXR)]
        for half in range(2):
            cgb = [pltpu.async_copy(
                       ab_ref.at[dv.at[half * (IDXR // 2) + j2]],
                       gb.at[pl.ds(j2 * IDXW, IDXW)], sem2)
                   for j2 in range(IDXR // 2)]
            if half == 0:
                for c in cga:
                    c.wait()
            for c in cgb:
                c.wait()
            hbase = half * (SUB // 2)

            def row(i, _):
                for k in range(HID // 16):
                    s = pl.ds(k * 16, 16)
                    v = ga[hbase + i, s] + gb[i, s] + c1v[s]
                    ga[hbase + i, s] = jnp.maximum(v, 0.0)
                return _

            lax.fori_loop(0, SUB // 2, row, None)
        pltpu.sync_copy(ga, r_ref.at[pl.ds(base, SUB)])


def _sc_mlpg(src2d, dstn2d, ab_flat, c1):
    f = pl.kernel(
        _mlpg_body,
        out_type=jax.ShapeDtypeStruct((E, HID), _f32),
        mesh=_MESH,
        compiler_params=_SCPARAMS,
        scratch_types=[
            pltpu.VMEM((IDXR, IDXW), _i32),
            pltpu.VMEM((IDXR, IDXW), _i32),
            pltpu.VMEM((SUB, HID), _f32),
            pltpu.VMEM((SUB // 2, HID), _f32),
            pltpu.VMEM((HID,), _f32),
            pltpu.SemaphoreType.DMA,
            pltpu.SemaphoreType.DMA,
        ],
    )
    return f(src2d, dstn2d, ab_flat, c1)


def kernel(x, edge_index, edge_attr, emb, W1, We1, as1, ad1, ae1, b1,
           W2, We2, as2, ad2, ae2, b2, M1, c1, M2, c2):
    src = edge_index[0]
    dst = edge_index[1]
    src2d = src.reshape(E // IDXW, IDXW)
    dst2d = dst.reshape(E // IDXW, IDXW)

    w1r = W1.reshape(D, HEADS, HID)
    W1sd = jnp.concatenate([
        jnp.einsum("dhc,hc->dh", w1r, as1),
        jnp.einsum("dhc,hc->dh", w1r, ad1)], axis=1)
    Wae = jnp.zeros((DE, 16), _f32)
    Wae = Wae.at[:, 0:HEADS].set(
        jnp.einsum("dhc,hc->dh", We1.reshape(DE, HEADS, HID), ae1))
    Wae = Wae.at[:, 8].set(We2 @ ae2[0])
    W2sd = jnp.zeros((HEADS * HID, 16), _f32)
    W2sd = W2sd.at[:, 0].set(W2 @ as2[0]).at[:, 1].set(W2 @ ad2[0])
    w1hm = w1r.transpose(1, 0, 2)
    w2hm = W2.reshape(HEADS, HID, HID)
    w2sdhm = W2sd.reshape(HEADS, HID, 16)
    b1hm = b1.reshape(HEADS, 1, HID)
    m2p = jnp.zeros((HID, 8), _f32).at[:, 0:3].set(M2)
    c2p = jnp.zeros((8,), _f32).at[0:3].set(c2)

    heads_off = (jnp.arange(HEADS, dtype=_i32) * N)[:, None, None]
    gidx1 = (src2d[None] + heads_off).astype(_i32)
    gidx2 = src2d[None]
    dstn2d = dst2d + N

    zeros16 = jnp.zeros((N, 16), _f32)
    zeros64 = jnp.zeros((N, HID), _f32)

    xh1hm, sd1 = _tc_node1(x.astype(_i32), emb, w1hm, W1sd)
    eatt = _tc_eatt(edge_attr, Wae)
    ex1 = _sc_att(src2d, dst2d, sd1, eatt, 1)
    den1, out1p = _sc_agg(gidx1, dst2d, ex1,
                          xh1hm.reshape(HEADS * N, HID), zeros16, zeros64,
                          HEADS)
    out1p = out1p.reshape(NC, HEADS, N, HID)

    xh2, sd2 = _tc_h1xh2(out1p, den1, b1hm, w2hm, w2sdhm)
    ex2 = _sc_att(src2d, dst2d, sd2, eatt, 2)
    den2, out2p = _sc_agg(gidx2, dst2d, ex2, xh2, zeros16, zeros64, 1)
    out2p = out2p.reshape(NC, 1, N, HID)

    ab = _tc_ab(out2p, den2, b2, M1[:HID], M1[HID:])
    r = _sc_mlpg(src2d, dstn2d, ab.reshape(2 * N, HID), c1)
    out = _tc_mlp(r, m2p, c2p)
    return out[:, 0:3]
